# Initial kernel scaffold; baseline (speedup 1.0000x reference)
import jax
import jax.numpy as jnp
from jax.experimental import pallas as pl


def kernel(input_ids, word_emb, pos_emb, ln_scale, ln_bias, kernel, bias):
    raise NotImplementedError("write your pallas kernel here")


# speedup vs baseline: 4.0732x; 4.0732x over previous
import functools

import jax
import jax.numpy as jnp
from jax import lax
from jax.experimental import pallas as pl
from jax.experimental.pallas import tpu as pltpu
from jax.experimental.pallas import tpu_sc as plsc

LN_EPS = 1e-12

_NC = 2
_NS = 16
_NW = _NC * _NS

_CHUNK = 128


def _sc_gather(ids3, table):
  nw, n_ch, chunk = ids3.shape
  e = table.shape[1]
  n = nw * n_ch * chunk
  rows_per_w = n_ch * chunk
  mesh = plsc.VectorSubcoreMesh(core_axis_name="c", subcore_axis_name="s")

  @functools.partial(
      pl.kernel,
      mesh=mesh,
      out_type=jax.ShapeDtypeStruct((n, e), jnp.float32),
      scratch_types=[
          pltpu.VMEM((n_ch, chunk), jnp.int32),
          pltpu.VMEM((chunk, e), jnp.float32),
          pltpu.SemaphoreType.DMA,
      ],
  )
  def gather_kernel(ids_hbm, tab_hbm, out_hbm, idx_v, rows_v, sem):
    wid = lax.axis_index("s") * _NC + lax.axis_index("c")
    base = wid * rows_per_w
    pltpu.sync_copy(ids_hbm.at[wid], idx_v)

    def body(i, carry):
      pltpu.async_copy(tab_hbm.at[idx_v.at[i]], rows_v, sem).wait()
      pltpu.sync_copy(rows_v, out_hbm.at[pl.ds(base + i * chunk, chunk)])
      return carry

    lax.fori_loop(0, n_ch, body, 0)

  return gather_kernel(ids3, table)


def _tc_fused(x, pos_tiled, g, b, proj, bias):
  n, e = x.shape
  r = pos_tiled.shape[0]
  h = proj.shape[1]

  def body(x_ref, p_ref, g_ref, b_ref, k_ref, bias_ref, o_ref):
    xv = x_ref[...] + p_ref[...]
    mu = jnp.mean(xv, axis=1, keepdims=True)
    xc = xv - mu
    var = jnp.mean(xc * xc, axis=1, keepdims=True)
    y = xc * lax.rsqrt(var + LN_EPS) * g_ref[...] + b_ref[...]
    o_ref[...] = (
        jnp.dot(y, k_ref[...], preferred_element_type=jnp.float32)
        + bias_ref[...]
    )

  return pl.pallas_call(
      body,
      grid=(n // r,),
      in_specs=[
          pl.BlockSpec((r, e), lambda i: (i, 0)),
          pl.BlockSpec((r, e), lambda i: (0, 0)),
          pl.BlockSpec((1, e), lambda i: (0, 0)),
          pl.BlockSpec((1, e), lambda i: (0, 0)),
          pl.BlockSpec((e, h), lambda i: (0, 0)),
          pl.BlockSpec((1, h), lambda i: (0, 0)),
      ],
      out_specs=pl.BlockSpec((r, h), lambda i: (i, 0)),
      out_shape=jax.ShapeDtypeStruct((n, h), jnp.float32),
  )(x, pos_tiled, g, b, proj, bias)


def kernel(input_ids, word_emb, pos_emb, ln_scale, ln_bias, kernel, bias):
  bsz, seq = input_ids.shape
  n = bsz * seq
  ids3 = input_ids.astype(jnp.int32).reshape(_NW, n // (_NW * _CHUNK), _CHUNK)
  gathered = _sc_gather(ids3, word_emb)

  seqs_per_block = 8
  pos_tiled = jnp.tile(pos_emb[:seq], (seqs_per_block, 1))
  out = _tc_fused(
      gathered,
      pos_tiled,
      ln_scale[None, :],
      ln_bias[None, :],
      kernel,
      bias[None, :],
  )
  return out.reshape(bsz, seq, -1)


# speedup vs baseline: 4.1721x; 1.0243x over previous
import functools

import jax
import jax.numpy as jnp
from jax import lax
from jax.experimental import pallas as pl
from jax.experimental.pallas import tpu as pltpu
from jax.experimental.pallas import tpu_sc as plsc

LN_EPS = 1e-12

_NC = 2
_NS = 16
_NW = _NC * _NS

_GROUPS = 4
_CHUNK = 80
_SEQS_PER_BLOCK = 8


def _sc_gather(ids3, table):
  nw, n_ch, chunk = ids3.shape
  e = table.shape[1]
  n = nw * n_ch * chunk
  rows_per_w = n_ch * chunk
  mesh = plsc.VectorSubcoreMesh(core_axis_name="c", subcore_axis_name="s")

  @functools.partial(
      pl.kernel,
      mesh=mesh,
      out_type=jax.ShapeDtypeStruct((n, e), jnp.float32),
      scratch_types=[
          pltpu.VMEM((n_ch, chunk), jnp.int32),
          pltpu.VMEM((chunk, e), jnp.float32),
          pltpu.SemaphoreType.DMA,
      ],
  )
  def gather_kernel(ids_hbm, tab_hbm, out_hbm, idx_v, rows_v, sem):
    wid = lax.axis_index("s") * _NC + lax.axis_index("c")
    base = wid * rows_per_w
    pltpu.sync_copy(ids_hbm.at[wid], idx_v)

    def body(i, carry):
      pltpu.async_copy(tab_hbm.at[idx_v.at[i]], rows_v, sem).wait()
      pltpu.sync_copy(rows_v, out_hbm.at[pl.ds(base + i * chunk, chunk)])
      return carry

    lax.fori_loop(0, n_ch, body, 0)

  return gather_kernel(ids3, table)


def _tc_fused_group(x, pos_tiled, g, b, proj, bias, grp, n_total, out_buf):
  rows, e = x.shape
  r = pos_tiled.shape[0]
  h = proj.shape[1]
  grid_n = rows // r
  row0 = grp * grid_n

  def body(*refs):
    x_ref, p_ref, g_ref, b_ref, k_ref, bias_ref = refs[:6]
    o_ref = refs[-1]
    xv = x_ref[...] + p_ref[...]
    mu = jnp.mean(xv, axis=1, keepdims=True)
    xc = xv - mu
    var = jnp.mean(xc * xc, axis=1, keepdims=True)
    y = xc * lax.rsqrt(var + LN_EPS) * g_ref[...] + b_ref[...]
    o_ref[...] = (
        jnp.dot(y, k_ref[...], preferred_element_type=jnp.float32)
        + bias_ref[...]
    )

  in_specs = [
      pl.BlockSpec((r, e), lambda i: (i, 0)),
      pl.BlockSpec((r, e), lambda i: (0, 0)),
      pl.BlockSpec((1, e), lambda i: (0, 0)),
      pl.BlockSpec((1, e), lambda i: (0, 0)),
      pl.BlockSpec((e, h), lambda i: (0, 0)),
      pl.BlockSpec((1, h), lambda i: (0, 0)),
  ]
  args = [x, pos_tiled, g, b, proj, bias]
  aliases = {}
  if out_buf is not None:
    in_specs.append(pl.BlockSpec(memory_space=pl.ANY))
    args.append(out_buf)
    aliases = {6: 0}

  return pl.pallas_call(
      body,
      grid=(grid_n,),
      in_specs=in_specs,
      out_specs=pl.BlockSpec((r, h), lambda i: (row0 + i, 0)),
      out_shape=jax.ShapeDtypeStruct((n_total, h), jnp.float32),
      input_output_aliases=aliases,
  )(*args)


def kernel(input_ids, word_emb, pos_emb, ln_scale, ln_bias, kernel, bias):
  bsz, seq = input_ids.shape
  n = bsz * seq
  rows_g = n // _GROUPS
  ids4 = input_ids.astype(jnp.int32).reshape(
      _GROUPS, _NW, rows_g // (_NW * _CHUNK), _CHUNK
  )

  gathered = [_sc_gather(ids4[grp], word_emb) for grp in range(_GROUPS)]

  pos_tiled = jnp.tile(pos_emb[:seq], (_SEQS_PER_BLOCK, 1))
  g2 = ln_scale[None, :]
  b2 = ln_bias[None, :]
  bias2 = bias[None, :]
  out = None
  for grp in range(_GROUPS):
    out = _tc_fused_group(
        gathered[grp], pos_tiled, g2, b2, kernel, bias2, grp, n, out
    )
  return out.reshape(bsz, seq, -1)
